# baseline (speedup 1.0000x reference)
import jax
import jax.numpy as jnp
from jax import lax
from jax.experimental import pallas as pl
from jax.experimental.pallas import tpu as pltpu
from jax.experimental.pallas import tpu_sc as plsc

B, C, H, W, P = 64, 3, 512, 512, 16
N = 716
D = C * P * P
SEG = D // 16
ROWS_PER_B = C * H * W // 16
NC, NS = 2, 16
NW = NC * NS
NP = 24
CHUNK = 96
NCHUNK = NP * SEG // CHUNK
UNROLL = 8


def _sc_body(table_hbm, idx_hbm, mask_hbm, out_hbm, idx_v, mask_v, data_v, sem):
    wid = lax.axis_index("s") * NC + lax.axis_index("c")
    i0 = jnp.minimum(wid * NP, N - NP)
    pltpu.sync_copy(mask_hbm.at[pl.ds(i0 * SEG, NP * SEG)], mask_v)

    def body(b, carry):
        pltpu.sync_copy(idx_hbm.at[b * NW + wid], idx_v)
        copies = [
            pltpu.async_copy(
                table_hbm.at[idx_v.at[k]],
                data_v.at[pl.ds(k * CHUNK, CHUNK)],
                sem,
            )
            for k in range(NCHUNK)
        ]
        for cp in copies:
            cp.wait()

        def mul(r, c2):
            for u in range(UNROLL):
                rr = r * UNROLL + u
                data_v[rr, :] = data_v[rr, :] * mask_v[rr, :]
            return c2

        lax.fori_loop(0, NP * SEG // UNROLL, mul, 0)
        pltpu.sync_copy(data_v, out_hbm.at[pl.ds((b * N + i0) * SEG, NP * SEG)])
        return carry

    lax.fori_loop(0, B, body, 0)


def kernel(images, patch_indices, patch_mask):
    table = images.reshape(B * ROWS_PER_B, 16)
    h = patch_indices // (W // P)
    w = patch_indices % (W // P)
    off_i = (h * (W // P) * P + w).astype(jnp.int32)
    j = jnp.arange(SEG, dtype=jnp.int32)
    off_j = (j // P) * (H * W // 16) + (j % P) * (W // P)
    starts = jnp.minimum(jnp.arange(NW, dtype=jnp.int32) * NP, N - NP)
    rows = starts[:, None] + jnp.arange(NP, dtype=jnp.int32)[None, :]
    base = off_i[rows][:, :, None] + off_j[None, None, :]
    idx = (
        base.reshape(1, NW, NP * SEG)
        + (jnp.arange(B, dtype=jnp.int32) * ROWS_PER_B)[:, None, None]
    ).reshape(B * NW, NCHUNK, CHUNK)
    mask_f = patch_mask.astype(jnp.float32).reshape(N * SEG, 16)

    run = pl.kernel(
        _sc_body,
        out_type=jax.ShapeDtypeStruct((B * N * SEG, 16), jnp.float32),
        mesh=plsc.VectorSubcoreMesh(core_axis_name="c", subcore_axis_name="s"),
        compiler_params=pltpu.CompilerParams(use_tc_tiling_on_sc=False),
        scratch_types=[
            pltpu.VMEM((NCHUNK, CHUNK), jnp.int32),
            pltpu.VMEM((NP * SEG, 16), jnp.float32),
            pltpu.VMEM((NP * SEG, 16), jnp.float32),
            pltpu.SemaphoreType.DMA,
        ],
    )
    out = run(table, idx, mask_f)
    return out.reshape(B, N, D)
